# baseline (speedup 1.0000x reference)
import functools

import jax
import jax.numpy as jnp
from jax.experimental import pallas as pl
from jax.experimental.pallas import tpu as pltpu

_BN = 256


def _body(x_ref, a1_ref, a2_ref, ck_ref, cb_ref, dk_ref, db_ref, lk_ref,
          lb_ref, out_ref, h1_ref, sum_ref, max_ref, *, n_steps, f_in):
    j = pl.program_id(1)
    bn = a1_ref.shape[1]

    @pl.when(j == 0)
    def _init():
        h1_ref[...] = jnp.dot(x_ref[0, :, :f_in], ck_ref[...],
                              preferred_element_type=jnp.float32)

    z1 = jnp.dot(a1_ref[0], h1_ref[...], preferred_element_type=jnp.float32)
    z2 = jnp.dot(a2_ref[0], h1_ref[...], preferred_element_type=jnp.float32)

    m1 = x_ref[0, pl.ds(j * 2 * bn, bn), f_in:f_in + 1] != 0.0
    m2 = x_ref[0, pl.ds(j * 2 * bn + bn, bn), f_in:f_in + 1] != 0.0
    zsum = jnp.where(m1, z1, 0.0) + jnp.where(m2, z2, 0.0)
    zmax = jnp.maximum(jnp.where(m1, z1, -jnp.inf),
                       jnp.where(m2, z2, -jnp.inf))

    @pl.when(j == 0)
    def _first():
        sum_ref[...] = zsum
        max_ref[...] = zmax

    @pl.when(j > 0)
    def _rest():
        sum_ref[...] = sum_ref[...] + zsum
        max_ref[...] = jnp.maximum(max_ref[...], zmax)

    @pl.when(j == n_steps - 1)
    def _final():
        mall = x_ref[0, :, f_in:f_in + 1] != 0.0
        cnt = jnp.sum(mall.astype(jnp.float32))
        ssum = jnp.sum(sum_ref[...], axis=0, keepdims=True)
        smax = jnp.max(max_ref[...], axis=0, keepdims=True)
        avg = ssum / jnp.maximum(cnt, 1.0) + cb_ref[...] * jnp.minimum(cnt, 1.0)
        smax = jnp.maximum(smax + cb_ref[...], -1e9)
        pooled = jnp.concatenate([avg, smax], axis=1)
        hid = jnp.dot(pooled, dk_ref[...],
                      preferred_element_type=jnp.float32) + db_ref[...]
        hid = jnp.maximum(hid, 0.0)
        out = jnp.dot(hid, lk_ref[...],
                      preferred_element_type=jnp.float32) + lb_ref[...]
        out_ref[0] = out


@jax.jit
def kernel(x, a, conv1_kernel, conv1_bias, dense1_kernel, dense1_bias,
           last_kernel, last_bias):
    B, N, fp1 = x.shape
    f_in = fp1 - 1
    hdim = conv1_kernel.shape[1]
    n_hidden = dense1_kernel.shape[1]
    n_labels = last_kernel.shape[1]
    bn = _BN
    n_steps = N // (2 * bn)

    cb = conv1_bias.reshape(1, hdim)
    db = dense1_bias.reshape(1, n_hidden)
    lb = last_bias.reshape(1, n_labels)

    grid = (B, n_steps)
    out = pl.pallas_call(
        functools.partial(_body, n_steps=n_steps, f_in=f_in),
        grid=grid,
        in_specs=[
            pl.BlockSpec((1, N, fp1), lambda b, j: (b, 0, 0)),
            pl.BlockSpec((1, bn, N), lambda b, j: (b, 2 * j, 0)),
            pl.BlockSpec((1, bn, N), lambda b, j: (b, 2 * j + 1, 0)),
            pl.BlockSpec((f_in, hdim), lambda b, j: (0, 0)),
            pl.BlockSpec((1, hdim), lambda b, j: (0, 0)),
            pl.BlockSpec((2 * hdim, n_hidden), lambda b, j: (0, 0)),
            pl.BlockSpec((1, n_hidden), lambda b, j: (0, 0)),
            pl.BlockSpec((n_hidden, n_labels), lambda b, j: (0, 0)),
            pl.BlockSpec((1, n_labels), lambda b, j: (0, 0)),
        ],
        out_specs=pl.BlockSpec((1, 1, n_labels), lambda b, j: (b, 0, 0)),
        out_shape=jax.ShapeDtypeStruct((B, 1, n_labels), jnp.float32),
        scratch_shapes=[
            pltpu.VMEM((N, hdim), jnp.float32),
            pltpu.VMEM((bn, hdim), jnp.float32),
            pltpu.VMEM((bn, hdim), jnp.float32),
        ],
        compiler_params=pltpu.CompilerParams(
            dimension_semantics=("arbitrary", "arbitrary"),
        ),
    )(x, a, a, conv1_kernel, cb, dense1_kernel, db, last_kernel, lb)
    return out.reshape(B, n_labels)


# speedup vs baseline: 1.1560x; 1.1560x over previous
import functools

import jax
import jax.numpy as jnp
from jax.experimental import pallas as pl
from jax.experimental.pallas import tpu as pltpu

_BN = 512
_NBUF = 4


def _body(x_ref, a_hbm, ck_ref, cb_ref, dk_ref, db_ref, lk_ref, lb_ref,
          out_ref, h1_ref, sum_ref, max_ref, *, n_steps, f_in, n_b):
    n = x_ref.shape[1]
    bn = n // n_steps

    def _h1(b, _):
        h1_ref[b] = jnp.dot(x_ref[b, :, :f_in], ck_ref[...],
                            preferred_element_type=jnp.float32)
        return 0
    jax.lax.fori_loop(0, n_b, _h1, 0)

    def _inner(a_blk):
        b = pl.program_id(0)
        j = pl.program_id(1)

        z = jnp.dot(a_blk[0], h1_ref[b],
                    preferred_element_type=jnp.float32)
        m = x_ref[b, pl.ds(j * bn, bn), f_in:f_in + 1] != 0.0
        zsum = jnp.where(m, z, 0.0)
        zmax = jnp.where(m, z, -jnp.inf)

        @pl.when(j == 0)
        def _first():
            sum_ref[...] = zsum
            max_ref[...] = zmax

        @pl.when(j > 0)
        def _rest():
            sum_ref[...] = sum_ref[...] + zsum
            max_ref[...] = jnp.maximum(max_ref[...], zmax)

        @pl.when(j == n_steps - 1)
        def _final():
            mall = x_ref[b, :, f_in:f_in + 1] != 0.0
            cnt = jnp.sum(mall.astype(jnp.float32))
            ssum = jnp.sum(sum_ref[...], axis=0, keepdims=True)
            smax = jnp.max(max_ref[...], axis=0, keepdims=True)
            avg = (ssum / jnp.maximum(cnt, 1.0)
                   + cb_ref[...] * jnp.minimum(cnt, 1.0))
            smax2 = jnp.maximum(smax + cb_ref[...], -1e9)
            pooled = jnp.concatenate([avg, smax2], axis=1)
            hid = jnp.dot(pooled, dk_ref[...],
                          preferred_element_type=jnp.float32) + db_ref[...]
            hid = jnp.maximum(hid, 0.0)
            out = jnp.dot(hid, lk_ref[...],
                          preferred_element_type=jnp.float32) + lb_ref[...]
            out_ref[b] = out

    pipeline = pltpu.emit_pipeline(
        _inner,
        grid=(n_b, n_steps),
        in_specs=[
            pl.BlockSpec((1, bn, n), lambda b, j: (b, j, 0),
                         pipeline_mode=pl.Buffered(buffer_count=_NBUF,
                                                   use_lookahead=True)),
        ],
        out_specs=(),
    )
    pipeline(a_hbm)


@jax.jit
def kernel(x, a, conv1_kernel, conv1_bias, dense1_kernel, dense1_bias,
           last_kernel, last_bias):
    B, N, fp1 = x.shape
    f_in = fp1 - 1
    hdim = conv1_kernel.shape[1]
    n_hidden = dense1_kernel.shape[1]
    n_labels = last_kernel.shape[1]
    bn = _BN
    n_steps = N // bn

    cb = conv1_bias.reshape(1, hdim)
    db = dense1_bias.reshape(1, n_hidden)
    lb = last_bias.reshape(1, n_labels)

    vmem = pl.BlockSpec(memory_space=pltpu.VMEM)
    out = pl.pallas_call(
        functools.partial(_body, n_steps=n_steps, f_in=f_in, n_b=B),
        in_specs=[
            vmem,
            pl.BlockSpec(memory_space=pl.ANY),
            vmem, vmem, vmem, vmem, vmem, vmem,
        ],
        out_specs=pl.BlockSpec(memory_space=pltpu.VMEM),
        out_shape=jax.ShapeDtypeStruct((B, 1, n_labels), jnp.float32),
        scratch_shapes=[
            pltpu.VMEM((B, N, hdim), jnp.float32),
            pltpu.VMEM((bn, hdim), jnp.float32),
            pltpu.VMEM((bn, hdim), jnp.float32),
        ],
    )(x, a, conv1_kernel, cb, dense1_kernel, db, last_kernel, lb)
    return out.reshape(B, n_labels)
